# baseline (speedup 1.0000x reference)
import functools

import jax
import jax.numpy as jnp
from jax import lax
from jax.experimental import pallas as pl
from jax.experimental.pallas import tpu as pltpu
from jax.experimental.pallas import tpu_sc as plsc

_INFO = plsc.get_sparse_core_info()
_NC, _NS = _INFO.num_cores, _INFO.num_subcores
_NW = _NC * _NS

_CHUNK = 400
_NBUF = 4


@functools.lru_cache(maxsize=None)
def _build(B, V, D):
    b_per_w = B // _NW
    n_chunks = b_per_w // _CHUNK
    assert B % _NW == 0 and b_per_w % _CHUNK == 0 and n_chunks % _NBUF == 0
    mesh = plsc.VectorSubcoreMesh(core_axis_name="c", subcore_axis_name="s")

    @functools.partial(
        pl.kernel,
        mesh=mesh,
        out_type=jax.ShapeDtypeStruct((B, D), jnp.float32),
        scratch_types=[
            pltpu.VMEM((b_per_w,), jnp.int32),
            [pltpu.VMEM((_CHUNK, D), jnp.float32) for _ in range(_NBUF)],
            [pltpu.SemaphoreType.DMA for _ in range(_NBUF)],
            [pltpu.SemaphoreType.DMA for _ in range(_NBUF)],
        ],
        compiler_params=pltpu.CompilerParams(use_tc_tiling_on_sc=False),
    )
    def gather_kernel(table_hbm, idx_hbm, out_hbm, idx_v, rows, gsem, osem):
        wid = lax.axis_index("s") * _NC + lax.axis_index("c")
        base = wid * b_per_w

        pltpu.sync_copy(idx_hbm.at[pl.ds(base, b_per_w)], idx_v)

        def gather_copy(c, b):
            return pltpu.make_async_copy(
                table_hbm.at[idx_v.at[pl.ds(c * _CHUNK, _CHUNK)]],
                rows[b],
                gsem[b],
            )

        def out_copy(c, b):
            return pltpu.make_async_copy(
                rows[b],
                out_hbm.at[pl.ds(base + c * _CHUNK, _CHUNK)],
                osem[b],
            )

        for b in range(_NBUF - 1):
            gather_copy(b, b).start()

        def step(j, carry):
            for b in range(_NBUF):
                c = j * _NBUF + b
                gather_copy(c, b).wait()
                out_copy(c, b).start()
                pb = (b - 1) % _NBUF

                @pl.when(c > 0)
                def _():
                    out_copy(c - 1, pb).wait()

                @pl.when(c + _NBUF - 1 < n_chunks)
                def _():
                    gather_copy(c + _NBUF - 1, pb).start()

            return carry

        lax.fori_loop(0, n_chunks // _NBUF, step, 0, unroll=False)
        out_copy(n_chunks - 1, _NBUF - 1).wait()

    return gather_kernel


def kernel(x, peso):
    B0, S = x.shape
    V, D = peso.shape
    flat_idx = x.reshape(B0 * S)
    out = _build(B0 * S, V, D)(peso, flat_idx)
    return out.reshape(B0, S, D)
